# baseline (speedup 1.0000x reference)
import jax
import jax.numpy as jnp
from jax.experimental import pallas as pl


def _copy_body(src, dst):
    dst[...] = src[...]


def _tc_copy(x, max_grid=1):
    rows, cols = x.shape
    g = 1
    for cand in range(max_grid, 0, -1):
        if rows % cand == 0 and (rows // cand) % 8 == 0:
            g = cand
            break
    b = rows // g
    return pl.pallas_call(
        _copy_body,
        grid=(g,),
        in_specs=(pl.BlockSpec((b, cols), lambda i: (i, 0)),),
        out_specs=pl.BlockSpec((b, cols), lambda i: (i, 0)),
        out_shape=jax.ShapeDtypeStruct((rows, cols), x.dtype),
    )(x)


def kernel(node_latents_from, node_latents_to, edge_latents, edge_index, receivers_count):
    new_node_latents = _tc_copy(node_latents_to)
    return (new_node_latents, edge_latents)


# speedup vs baseline: 1.0464x; 1.0464x over previous
import jax
import jax.numpy as jnp
from jax.experimental import pallas as pl


def _copy_body(src, dst):
    dst[...] = src[...]


def _tc_copy(x, max_grid=2):
    rows, cols = x.shape
    g = 1
    for cand in range(max_grid, 0, -1):
        if rows % cand == 0 and (rows // cand) % 8 == 0:
            g = cand
            break
    b = rows // g
    return pl.pallas_call(
        _copy_body,
        grid=(g,),
        in_specs=(pl.BlockSpec((b, cols), lambda i: (i, 0)),),
        out_specs=pl.BlockSpec((b, cols), lambda i: (i, 0)),
        out_shape=jax.ShapeDtypeStruct((rows, cols), x.dtype),
    )(x)


def kernel(node_latents_from, node_latents_to, edge_latents, edge_index, receivers_count):
    new_node_latents = _tc_copy(node_latents_to)
    return (new_node_latents, edge_latents)
